# baseline (speedup 1.0000x reference)
import jax
import jax.numpy as jnp
from jax import lax
from jax.experimental import pallas as pl
from jax.experimental.pallas import tpu as pltpu
from jax.experimental.pallas import tpu_sc as plsc

N = 10000
H = 10000
E = 320000
D = 128

NC = 2
NS = 16
L = 16
QS = 2560
HS = 2 * QS
SP = NC * HS
RPTQ = QS // NS
RAW = 128
KB = 8
NBLK = 20
EPT = RAW * KB * NBLK
E_PAD = EPT * NS
ER = E_PAD // RAW
RPB = EPT // RAW
GC = 64
NBUF = 4
RING = NBUF * GC
CAP = 6144


def _phase_body(table_ref, init_ref, tcols_ref, trows_ref, tvals_ref,
                out_ref,
                cols_v, rows_v, vals_v, ccols, crows, cvals, sidx,
                rowbuf, acc, *sems):
    sem_g = sems[0:NBUF]
    sem_s = sems[NBUF:2 * NBUF]
    sem_i = sems[2 * NBUF:2 * NBUF + 2]
    c = lax.axis_index("c")
    s = lax.axis_index("s")
    lo = c * HS
    brow = s * RPB
    crefs = (tcols_ref, trows_ref, tvals_ref)

    def issue_idx(pb, blk):
        r = brow + blk * KB
        pltpu.async_copy(tcols_ref.at[pl.ds(r, KB)], cols_v.at[pb],
                         sem_i[pb])
        pltpu.async_copy(trows_ref.at[pl.ds(r, KB)], rows_v.at[pb],
                         sem_i[pb])
        pltpu.async_copy(tvals_ref.at[pl.ds(r, KB)], vals_v.at[pb],
                         sem_i[pb])

    def wait_idx(pb):
        for ref, buf in ((tcols_ref, cols_v), (trows_ref, rows_v),
                         (tvals_ref, vals_v)):
            pltpu.make_async_copy(ref.at[pl.ds(0, KB)], buf.at[pb],
                                  sem_i[pb]).wait()

    def wait_gather(j):
        pltpu.make_async_copy(table_ref.at[pl.ds(0, GC)], rowbuf.at[j],
                              sem_g[j]).wait()

    def wait_scatter(j):
        pltpu.make_async_copy(table_ref.at[pl.ds(0, GC)], rowbuf.at[j],
                              sem_s[j]).wait()

    full = jnp.ones((L,), jnp.bool_)

    issue_idx(0, 0)

    @pl.loop(0, 2)
    def quarter(qq):
        lo_q = lo + qq * QS

        def compact_block(pb, off):
            wait_idx(pb)

            @pl.loop(0, KB * RAW // L, init_carry=off)
            def grp(t, off):
                jr = t // (RAW // L)
                g = t % (RAW // L)
                cv = cols_v[pb, jr, pl.ds(g * L, L)]
                rv = rows_v[pb, jr, pl.ds(g * L, L)]
                vv = vals_v[pb, jr, pl.ds(g * L, L)]
                m = jnp.logical_and(rv >= lo_q, rv < lo_q + QS)
                cnt = plsc.all_reduce_population_count(m)[0]
                plsc.store_compressed(ccols.at[pl.ds(off, L)], cv, mask=m)
                plsc.store_compressed(crows.at[pl.ds(off, L)], rv - lo_q,
                                      mask=m)
                plsc.store_compressed(cvals.at[pl.ds(off, L)], vv, mask=m)
                return off + cnt

            return grp

        @pl.loop(0, NBLK, step=2, init_carry=jnp.int32(0))
        def blk_loop(blk, off):
            @pl.when(blk + 1 < NBLK)
            def _():
                issue_idx(1, blk + 1)
            off = compact_block(0, off)

            @pl.when(blk + 2 < NBLK)
            def _():
                issue_idx(0, blk + 2)
            off = compact_block(1, off)
            return off

        cnt = blk_loop

        @pl.when(qq == 0)
        def _():
            issue_idx(0, 0)

        zi = jnp.zeros((L,), jnp.int32)
        zf = jnp.zeros((L,), jnp.float32)

        @pl.loop(0, RING // L)
        def pad_loop(t):
            o = cnt + t * L
            plsc.store_compressed(ccols.at[pl.ds(o, L)], zi, mask=full)
            plsc.store_compressed(crows.at[pl.ds(o, L)], zi, mask=full)
            plsc.store_compressed(cvals.at[pl.ds(o, L)], zf, mask=full)

        pltpu.sync_copy(init_ref.at[pl.ds(lo_q + s * RPTQ, RPTQ)],
                        acc.at[pl.ds(s * RPTQ, RPTQ)])
        plsc.subcore_barrier()

        n_outer = (cnt + (RING - 1)) // RING

        @pl.loop(0, n_outer)
        def outer(o):
            for j in range(NBUF):
                base = (o * NBUF + j) * GC
                @pl.when(o > 0)
                def _():
                    wait_scatter(j)
                for g in range(GC // L):
                    sidx[j, pl.ds(g * L, L)] = crows[pl.ds(base + g * L, L)]
                pltpu.async_copy(table_ref.at[ccols.at[pl.ds(base, GC)]],
                                 rowbuf.at[j], sem_g[j])
            for j in range(NBUF):
                base = (o * NBUF + j) * GC
                wait_gather(j)

                @pl.loop(0, GC // L)
                def scale_group(g):
                    v16 = cvals[pl.ds(base + g * L, L)]
                    for e in range(L):
                        v = v16[e]
                        eidx = g * L + e
                        for q in range(D // L):
                            rowbuf[j, eidx, pl.ds(q * L, L)] = (
                                rowbuf[j, eidx, pl.ds(q * L, L)] * v)

                pltpu.async_copy(rowbuf.at[j], acc.at[sidx.at[j]],
                                 sem_s[j], add=True)

        for j in range(NBUF):
            @pl.when(n_outer > 0)
            def _():
                wait_scatter(j)
        plsc.subcore_barrier()

        pltpu.sync_copy(acc.at[pl.ds(s * RPTQ, RPTQ)],
                        out_ref.at[pl.ds(lo_q + s * RPTQ, RPTQ)])


def _make_phase():
    mesh = plsc.VectorSubcoreMesh(core_axis_name="c", subcore_axis_name="s",
                                  num_cores=NC, num_subcores=NS)
    return pl.kernel(
        _phase_body,
        out_type=jax.ShapeDtypeStruct((SP, D), jnp.float32),
        mesh=mesh,
        compiler_params=pltpu.CompilerParams(use_tc_tiling_on_sc=False,
                                             needs_layout_passes=False),
        scratch_types=[
            pltpu.VMEM((2, KB, RAW), jnp.int32),
            pltpu.VMEM((2, KB, RAW), jnp.int32),
            pltpu.VMEM((2, KB, RAW), jnp.float32),
            pltpu.VMEM((CAP,), jnp.int32),
            pltpu.VMEM((CAP,), jnp.int32),
            pltpu.VMEM((CAP,), jnp.float32),
            pltpu.VMEM((NBUF, GC), jnp.int32),
            pltpu.VMEM((NBUF, GC, D), jnp.float32),
            pltpu.VMEM_SHARED((QS, D), jnp.float32),
        ] + [pltpu.SemaphoreType.DMA] * (2 * NBUF + 2),
    )


_phase = _make_phase()


def _edges2d(a):
    return jnp.pad(a, (0, E_PAD - E)).reshape(ER, RAW)


def _rows2d(a):
    return jnp.pad(a, (0, E_PAD - E), constant_values=SP).reshape(ER, RAW)


@jax.jit
def kernel(x, src_rows, src_cols, src_vals, tar_rows, tar_cols, tar_vals):
    xp = jnp.pad(x, ((0, SP - N), (0, 0)))
    zeros = jnp.zeros((SP, D), jnp.float32)
    y2 = _phase(xp, zeros, _edges2d(tar_cols), _rows2d(tar_rows),
                _edges2d(tar_vals))
    out = _phase(y2, xp, _edges2d(src_cols), _rows2d(src_rows),
                 _edges2d(src_vals))
    return out[:N]


# speedup vs baseline: 1.2452x; 1.2452x over previous
import jax
import jax.numpy as jnp
from jax import lax
from jax.experimental import pallas as pl
from jax.experimental.pallas import tpu as pltpu
from jax.experimental.pallas import tpu_sc as plsc

N = 10000
H = 10000
E = 320000
D = 128

NC = 2
NS = 16
L = 16
DH = D // NC
CHUNK = 128
KB = 8
NBLK = 20
EPT = CHUNK * KB * NBLK
E_PAD = EPT * NS
ER = E_PAD // CHUNK
RPB = EPT // CHUNK
SP = 10240
RPT = SP // NS


def _fused_body(x2_ref, tcols_ref, trows_ref, tvals_ref,
                scols_ref, srows_ref, svals_ref, zeros_ref,
                y2_ref, out_ref,
                cols_v, rows_v, vals_v, rowbuf, acc, *sems):
    sem_g = sems[0:KB]
    sem_s = sems[KB:2 * KB]
    sem_i = sems[2 * KB:2 * KB + 2]
    c = lax.axis_index("c")
    s = lax.axis_index("s")
    col_off = c * SP
    brow = s * RPB

    def issue_idx(crefs, pb, blk):
        cref, rref, vref = crefs
        r = brow + blk * KB
        pltpu.async_copy(cref.at[pl.ds(r, KB)], cols_v.at[pb], sem_i[pb])
        pltpu.async_copy(rref.at[pl.ds(r, KB)], rows_v.at[pb], sem_i[pb])
        pltpu.async_copy(vref.at[pl.ds(r, KB)], vals_v.at[pb], sem_i[pb])

    def wait_idx(pb):
        for ref, buf in ((tcols_ref, cols_v), (trows_ref, rows_v),
                         (tvals_ref, vals_v)):
            pltpu.make_async_copy(ref.at[pl.ds(0, KB)], buf.at[pb],
                                  sem_i[pb]).wait()

    def wait_gather(j):
        pltpu.make_async_copy(x2_ref.at[pl.ds(0, CHUNK)], rowbuf.at[j],
                              sem_g[j]).wait()

    def wait_scatter(j):
        pltpu.make_async_copy(x2_ref.at[pl.ds(0, CHUNK)], rowbuf.at[j],
                              sem_s[j]).wait()

    def run_phase(table_ref, crefs, next_crefs):

        def process_block(blk, pb, guard_scatter_wait, tail_prefetch):
            wait_idx(pb)
            for jr in range(KB):
                for g in range(CHUNK // L):
                    cols_v[pb, jr, pl.ds(g * L, L)] = (
                        cols_v[pb, jr, pl.ds(g * L, L)] + col_off)
            for j in range(KB):
                if guard_scatter_wait:
                    @pl.when(blk > 0)
                    def _():
                        wait_scatter(j)
                else:
                    wait_scatter(j)
                pltpu.async_copy(table_ref.at[cols_v.at[pb, j]],
                                 rowbuf.at[j], sem_g[j])
            if tail_prefetch:
                @pl.when(blk + 1 < NBLK)
                def _():
                    issue_idx(crefs, 1 - pb, blk + 1)
                if next_crefs is not None:
                    @pl.when(blk + 1 >= NBLK)
                    def _():
                        issue_idx(next_crefs, 1 - pb, 0)
            else:
                issue_idx(crefs, 1 - pb, blk + 1)
            for j in range(KB):
                wait_gather(j)

                @pl.loop(0, CHUNK // L)
                def scale_group(g):
                    v16 = vals_v[pb, j, pl.ds(g * L, L)]
                    for e in range(L):
                        v = v16[e]
                        eidx = g * L + e
                        for q in range(DH // L):
                            rowbuf[j, eidx, pl.ds(q * L, L)] = (
                                rowbuf[j, eidx, pl.ds(q * L, L)] * v)

                pltpu.async_copy(rowbuf.at[j], acc.at[rows_v.at[pb, j]],
                                 sem_s[j], add=True)

        @pl.loop(0, NBLK, step=2)
        def blk_loop(blk):
            process_block(blk, 0, guard_scatter_wait=True,
                          tail_prefetch=False)
            process_block(blk + 1, 1, guard_scatter_wait=False,
                          tail_prefetch=True)

        for j in range(KB):
            wait_scatter(j)

    issue_idx((tcols_ref, trows_ref, tvals_ref), 0, 0)
    pltpu.sync_copy(zeros_ref.at[pl.ds(s * RPT, RPT)],
                    acc.at[pl.ds(s * RPT, RPT)])
    plsc.subcore_barrier()

    run_phase(x2_ref, (tcols_ref, trows_ref, tvals_ref),
              (scols_ref, srows_ref, svals_ref))

    plsc.subcore_barrier()
    pltpu.sync_copy(acc.at[pl.ds(s * RPT, RPT)],
                    y2_ref.at[pl.ds(c * SP + s * RPT, RPT)])
    pltpu.sync_copy(x2_ref.at[pl.ds(c * SP + s * RPT, RPT)],
                    acc.at[pl.ds(s * RPT, RPT)])
    plsc.subcore_barrier()

    run_phase(y2_ref, (scols_ref, srows_ref, svals_ref), None)
    plsc.subcore_barrier()
    pltpu.sync_copy(acc.at[pl.ds(s * RPT, RPT)],
                    out_ref.at[pl.ds(c * SP + s * RPT, RPT)])


def _make_fused():
    mesh = plsc.VectorSubcoreMesh(core_axis_name="c", subcore_axis_name="s",
                                  num_cores=NC, num_subcores=NS)
    return pl.kernel(
        _fused_body,
        out_type=(jax.ShapeDtypeStruct((NC * SP, DH), jnp.float32),
                  jax.ShapeDtypeStruct((NC * SP, DH), jnp.float32)),
        mesh=mesh,
        compiler_params=pltpu.CompilerParams(use_tc_tiling_on_sc=False),
        scratch_types=[
            pltpu.VMEM((2, KB, CHUNK), jnp.int32),
            pltpu.VMEM((2, KB, CHUNK), jnp.int32),
            pltpu.VMEM((2, KB, CHUNK), jnp.float32),
            pltpu.VMEM((KB, CHUNK, DH), jnp.float32),
            pltpu.VMEM_SHARED((SP, DH), jnp.float32),
        ] + [pltpu.SemaphoreType.DMA] * (2 * KB + 2),
    )


_fused = _make_fused()


def _edges2d(a):
    return jnp.pad(a, (0, E_PAD - E)).reshape(ER, CHUNK)


@jax.jit
def kernel(x, src_rows, src_cols, src_vals, tar_rows, tar_cols, tar_vals):
    pad_r = ((0, SP - N), (0, 0))
    x2 = jnp.concatenate([jnp.pad(x[:, :DH], pad_r),
                          jnp.pad(x[:, DH:], pad_r)], axis=0)
    zeros1 = jnp.zeros((SP, DH), jnp.float32)
    _, out2 = _fused(x2, _edges2d(tar_cols), _edges2d(tar_rows),
                     _edges2d(tar_vals), _edges2d(src_cols),
                     _edges2d(src_rows), _edges2d(src_vals), zeros1)
    return jnp.concatenate([out2[:N], out2[SP:SP + N]], axis=1)


# speedup vs baseline: 1.3597x; 1.0920x over previous
import jax
import jax.numpy as jnp
from jax import lax
from jax.experimental import pallas as pl
from jax.experimental.pallas import tpu as pltpu
from jax.experimental.pallas import tpu_sc as plsc

N = 10000
H = 10000
E = 320000
D = 128

NC = 2
NS = 16
L = 16
DH = D // NC
CHUNK = 128
KB = 8
NBLK = 20
EPT = CHUNK * KB * NBLK
E_PAD = EPT * NS
ER = E_PAD // CHUNK
RPB = EPT // CHUNK
SP = 10240
RPT = SP // NS


def _fused_body(x2_ref, tcols_ref, trows_ref, tvals_ref,
                scols_ref, srows_ref, svals_ref, zeros_ref,
                y2_ref, out_ref,
                cols_v, rows_v, vals_v, rowbuf, acc, *sems):
    sem_g = sems[0:KB]
    sem_s = sems[KB:2 * KB]
    sem_i = sems[2 * KB:2 * KB + 2]
    c = lax.axis_index("c")
    s = lax.axis_index("s")
    col_off = c * SP
    brow = s * RPB

    def issue_idx(crefs, pb, blk):
        cref, rref, vref = crefs
        r = brow + blk * KB
        pltpu.async_copy(cref.at[pl.ds(r, KB)], cols_v.at[pb], sem_i[pb])
        pltpu.async_copy(rref.at[pl.ds(r, KB)], rows_v.at[pb], sem_i[pb])
        pltpu.async_copy(vref.at[pl.ds(r, KB)], vals_v.at[pb], sem_i[pb])

    def wait_idx(pb):
        for ref, buf in ((tcols_ref, cols_v), (trows_ref, rows_v),
                         (tvals_ref, vals_v)):
            pltpu.make_async_copy(ref.at[pl.ds(0, KB)], buf.at[pb],
                                  sem_i[pb]).wait()

    def wait_gather(j):
        pltpu.make_async_copy(x2_ref.at[pl.ds(0, CHUNK)], rowbuf.at[j],
                              sem_g[j]).wait()

    def wait_scatter(j):
        pltpu.make_async_copy(x2_ref.at[pl.ds(0, CHUNK)], rowbuf.at[j],
                              sem_s[j]).wait()

    def run_phase(table_ref, crefs, next_crefs):

        def process_block(blk, pb, guard_scatter_wait, tail_prefetch):
            wait_idx(pb)
            for jr in range(KB):
                for g in range(CHUNK // L):
                    cols_v[pb, jr, pl.ds(g * L, L)] = (
                        cols_v[pb, jr, pl.ds(g * L, L)] + col_off)
            for j in range(KB):
                if guard_scatter_wait:
                    @pl.when(blk > 0)
                    def _():
                        wait_scatter(j)
                else:
                    wait_scatter(j)
                pltpu.async_copy(table_ref.at[cols_v.at[pb, j]],
                                 rowbuf.at[j], sem_g[j])
            if tail_prefetch:
                @pl.when(blk + 1 < NBLK)
                def _():
                    issue_idx(crefs, 1 - pb, blk + 1)
                if next_crefs is not None:
                    @pl.when(blk + 1 >= NBLK)
                    def _():
                        issue_idx(next_crefs, 1 - pb, 0)
            else:
                issue_idx(crefs, 1 - pb, blk + 1)
            for j in range(KB):
                wait_gather(j)

                @pl.loop(0, CHUNK // L, unroll=2)
                def scale_group(g):
                    v16 = vals_v[pb, j, pl.ds(g * L, L)]
                    for e in range(L):
                        v = v16[e]
                        eidx = g * L + e
                        for q in range(DH // L):
                            rowbuf[j, eidx, pl.ds(q * L, L)] = (
                                rowbuf[j, eidx, pl.ds(q * L, L)] * v)

                pltpu.async_copy(rowbuf.at[j], acc.at[rows_v.at[pb, j]],
                                 sem_s[j], add=True)

        @pl.loop(0, NBLK, step=2)
        def blk_loop(blk):
            process_block(blk, 0, guard_scatter_wait=True,
                          tail_prefetch=False)
            process_block(blk + 1, 1, guard_scatter_wait=False,
                          tail_prefetch=True)

        for j in range(KB):
            wait_scatter(j)

    issue_idx((tcols_ref, trows_ref, tvals_ref), 0, 0)
    pltpu.sync_copy(zeros_ref.at[pl.ds(s * RPT, RPT)],
                    acc.at[pl.ds(s * RPT, RPT)])
    plsc.subcore_barrier()

    run_phase(x2_ref, (tcols_ref, trows_ref, tvals_ref),
              (scols_ref, srows_ref, svals_ref))

    plsc.subcore_barrier()
    pltpu.sync_copy(acc.at[pl.ds(s * RPT, RPT)],
                    y2_ref.at[pl.ds(c * SP + s * RPT, RPT)])
    pltpu.sync_copy(x2_ref.at[pl.ds(c * SP + s * RPT, RPT)],
                    acc.at[pl.ds(s * RPT, RPT)])
    plsc.subcore_barrier()

    run_phase(y2_ref, (scols_ref, srows_ref, svals_ref), None)
    plsc.subcore_barrier()
    pltpu.sync_copy(acc.at[pl.ds(s * RPT, RPT)],
                    out_ref.at[pl.ds(c * SP + s * RPT, RPT)])


def _make_fused():
    mesh = plsc.VectorSubcoreMesh(core_axis_name="c", subcore_axis_name="s",
                                  num_cores=NC, num_subcores=NS)
    return pl.kernel(
        _fused_body,
        out_type=(jax.ShapeDtypeStruct((NC * SP, DH), jnp.float32),
                  jax.ShapeDtypeStruct((NC * SP, DH), jnp.float32)),
        mesh=mesh,
        compiler_params=pltpu.CompilerParams(use_tc_tiling_on_sc=False),
        scratch_types=[
            pltpu.VMEM((2, KB, CHUNK), jnp.int32),
            pltpu.VMEM((2, KB, CHUNK), jnp.int32),
            pltpu.VMEM((2, KB, CHUNK), jnp.float32),
            pltpu.VMEM((KB, CHUNK, DH), jnp.float32),
            pltpu.VMEM_SHARED((SP, DH), jnp.float32),
        ] + [pltpu.SemaphoreType.DMA] * (2 * KB + 2),
    )


_fused = _make_fused()


def _edges2d(a):
    return jnp.pad(a, (0, E_PAD - E)).reshape(ER, CHUNK)


@jax.jit
def kernel(x, src_rows, src_cols, src_vals, tar_rows, tar_cols, tar_vals):
    pad_r = ((0, SP - N), (0, 0))
    x2 = jnp.concatenate([jnp.pad(x[:, :DH], pad_r),
                          jnp.pad(x[:, DH:], pad_r)], axis=0)
    zeros1 = jnp.zeros((SP, DH), jnp.float32)
    _, out2 = _fused(x2, _edges2d(tar_cols), _edges2d(tar_rows),
                     _edges2d(tar_vals), _edges2d(src_cols),
                     _edges2d(src_rows), _edges2d(src_vals), zeros1)
    return jnp.concatenate([out2[:N], out2[SP:SP + N]], axis=1)
